# Initial kernel scaffold; baseline (speedup 1.0000x reference)
import jax
import jax.numpy as jnp
from jax.experimental import pallas as pl


def kernel(hidden_states, Wq, Wk, Wv, Wo, Wg, W_gate, W_up, W_down, ln1_w, ln1_b, ln2_w, ln2_b):
    raise NotImplementedError("write your pallas kernel here")


# speedup vs baseline: 1.3753x; 1.3753x over previous
import functools

import jax
import jax.numpy as jnp
from jax import lax
from jax.experimental import pallas as pl
from jax.experimental.pallas import tpu as pltpu
from jax.experimental.pallas import tpu_sc as plsc

B, S, H = 1, 2048, 1024
NH, HD = 16, 64
E, K = 8, 2
FF = 2048

BQ = 256
BS = 256
BM = 128
NBLK = 40
LPAD = NBLK * BM
CH = 128
NEG = -1e9
EPS = 1e-5

_bf16 = jnp.bfloat16
_f32 = jnp.float32


def _ln(x, w, b):
    mu = jnp.mean(x, axis=1, keepdims=True)
    var = jnp.mean((x - mu) ** 2, axis=1, keepdims=True)
    return (x - mu) / jnp.sqrt(var + EPS) * w + b


def _qkv_body(x_ref, wq_ref, wk_ref, wv_ref, g_ref, b_ref, q_ref, k_ref, v_ref):
    xn = _ln(x_ref[...], g_ref[...], b_ref[...]).astype(_bf16)
    q_ref[...] = jnp.dot(xn, wq_ref[...].astype(_bf16), preferred_element_type=_f32)
    k_ref[...] = jnp.dot(xn, wk_ref[...].astype(_bf16), preferred_element_type=_f32)
    v_ref[...] = jnp.dot(xn, wv_ref[...].astype(_bf16), preferred_element_type=_f32)


def _qkv_call(x, Wq, Wk, Wv, g, b):
    sd = jax.ShapeDtypeStruct((S, H), _f32)
    return pl.pallas_call(
        _qkv_body,
        grid=(S // BS,),
        in_specs=[
            pl.BlockSpec((BS, H), lambda i: (i, 0)),
            pl.BlockSpec((H, H), lambda i: (0, 0)),
            pl.BlockSpec((H, H), lambda i: (0, 0)),
            pl.BlockSpec((H, H), lambda i: (0, 0)),
            pl.BlockSpec((1, H), lambda i: (0, 0)),
            pl.BlockSpec((1, H), lambda i: (0, 0)),
        ],
        out_specs=[pl.BlockSpec((BS, H), lambda i: (i, 0))] * 3,
        out_shape=[sd, sd, sd],
    )(x, Wq, Wk, Wv, g, b)


def _rot_half(x):
    return jnp.concatenate([-x[:, HD // 2:], x[:, :HD // 2]], axis=1)


def _attn_body(q_ref, k_ref, v_ref, cq_ref, sq_ref, ck_ref, sk_ref, o_ref, kr_ref):
    i = pl.program_id(1)

    @pl.when(i == 0)
    def _():
        k = k_ref[0]
        kr_ref[...] = k * ck_ref[...] + _rot_half(k) * sk_ref[...]

    q = q_ref[0]
    qr = q * cq_ref[...] + _rot_half(q) * sq_ref[...]
    s = lax.dot_general(
        qr.astype(_bf16), kr_ref[...].astype(_bf16),
        (((1,), (1,)), ((), ())), preferred_element_type=_f32) * (1.0 / (HD ** 0.5))
    rows = i * BQ + lax.broadcasted_iota(jnp.int32, (BQ, S), 0)
    cols = lax.broadcasted_iota(jnp.int32, (BQ, S), 1)
    s = jnp.where(cols <= rows, s, NEG)
    m = jnp.max(s, axis=1, keepdims=True)
    p = jnp.exp(s - m)
    p = p / jnp.sum(p, axis=1, keepdims=True)
    o_ref[0] = jnp.dot(p.astype(_bf16), v_ref[0].astype(_bf16),
                       preferred_element_type=_f32)


def _attn_call(q, k, v, cos, sin):
    return pl.pallas_call(
        _attn_body,
        grid=(NH, S // BQ),
        in_specs=[
            pl.BlockSpec((1, BQ, HD), lambda h, i: (h, i, 0)),
            pl.BlockSpec((1, S, HD), lambda h, i: (h, 0, 0)),
            pl.BlockSpec((1, S, HD), lambda h, i: (h, 0, 0)),
            pl.BlockSpec((BQ, HD), lambda h, i: (i, 0)),
            pl.BlockSpec((BQ, HD), lambda h, i: (i, 0)),
            pl.BlockSpec((S, HD), lambda h, i: (0, 0)),
            pl.BlockSpec((S, HD), lambda h, i: (0, 0)),
        ],
        out_specs=pl.BlockSpec((1, BQ, HD), lambda h, i: (h, i, 0)),
        out_shape=jax.ShapeDtypeStruct((NH, S, HD), _f32),
        scratch_shapes=[pltpu.VMEM((S, HD), _f32)],
    )(q, k, v, cos, sin, cos, sin)


def _oproj_body(a_ref, wo_ref, h_ref, o_ref):
    o_ref[...] = h_ref[...] + jnp.dot(
        a_ref[...].astype(_bf16), wo_ref[...].astype(_bf16),
        preferred_element_type=_f32)


def _oproj_call(attn, Wo, x0):
    return pl.pallas_call(
        _oproj_body,
        grid=(S // BS,),
        in_specs=[
            pl.BlockSpec((BS, H), lambda i: (i, 0)),
            pl.BlockSpec((H, H), lambda i: (0, 0)),
            pl.BlockSpec((BS, H), lambda i: (i, 0)),
        ],
        out_specs=pl.BlockSpec((BS, H), lambda i: (i, 0)),
        out_shape=jax.ShapeDtypeStruct((S, H), _f32),
    )(attn, Wo, x0)


def _router_body(h_ref, wg_ref, g_ref, b_ref,
                 xn_ref, w_ref, dest_ref, blk_ref, aux_ref, rank_ref, a_ref):
    xn = _ln(h_ref[...], g_ref[...], b_ref[...])
    xn_ref[...] = xn
    logits = jnp.dot(xn.astype(_bf16), wg_ref[...].astype(_bf16),
                     preferred_element_type=_f32)
    mx = jnp.max(logits, axis=1, keepdims=True)
    p = jnp.exp(logits - mx)
    p = p / jnp.sum(p, axis=1, keepdims=True)

    io8 = lax.broadcasted_iota(jnp.int32, (S, E), 1)
    m1 = jnp.max(p, axis=1, keepdims=True)
    a1 = jnp.min(jnp.where(p == m1, io8, E), axis=1, keepdims=True)
    p2 = jnp.where(io8 == a1, -1.0, p)
    m2 = jnp.max(p2, axis=1, keepdims=True)
    a2 = jnp.min(jnp.where(p2 == m2, io8, E), axis=1, keepdims=True)
    sw = m1 + m2
    w_ref[:, 0:1] = m1 / sw
    w_ref[:, 1:2] = m2 / sw

    A = jnp.concatenate([a1, a2], axis=0)
    a_ref[...] = A

    ri = lax.broadcasted_iota(jnp.int32, (CH, CH), 0)
    ci = lax.broadcasted_iota(jnp.int32, (CH, CH), 1)
    TI = (ci <= ri).astype(_bf16)

    def chunk(c, offs):
        ec = a_ref[pl.ds(c * CH, CH), :]
        oh = (lax.broadcasted_iota(jnp.int32, (CH, E), 1) == ec).astype(_f32)
        cum = jnp.dot(TI, oh.astype(_bf16), preferred_element_type=_f32)
        rank = jnp.sum(oh * (cum - 1.0), axis=1, keepdims=True)
        gsel = jnp.sum(oh * offs, axis=1, keepdims=True)
        rank_ref[pl.ds(c * CH, CH), :] = rank + gsel
        return offs + cum[CH - 1:CH, :]

    counts = lax.fori_loop(0, 2 * S // CH, chunk, jnp.zeros((1, E), _f32))

    padded = jnp.floor((counts + (BM - 1.0)) * (1.0 / BM)) * BM
    iu = lax.broadcasted_iota(jnp.int32, (E, E), 0)
    ju = lax.broadcasted_iota(jnp.int32, (E, E), 1)
    U = (iu < ju).astype(_f32)
    pb = jnp.dot(padded, U, precision=lax.Precision.HIGHEST,
                 preferred_element_type=_f32)
    pad_end = pb + padded

    jv = lax.broadcasted_iota(jnp.int32, (1, 64), 1).astype(_f32) * BM
    cnt = jnp.zeros((1, 64), _f32)
    for e in range(E):
        cnt = cnt + (pad_end[0:1, e:e + 1] <= jv).astype(_f32)
    blk = jnp.minimum(cnt, E - 1.0).astype(jnp.int32)
    blk_ref[...] = jnp.broadcast_to(blk, (8, 64))

    ohA = (lax.broadcasted_iota(jnp.int32, (2 * S, E), 1) == A).astype(_f32)
    psel = jnp.sum(ohA * pb, axis=1, keepdims=True)
    dall = (rank_ref[...] + psel).astype(jnp.int32)
    dest_ref[:, 0:1] = dall[:S]
    dest_ref[:, 1:2] = dall[S:]

    pmean = jnp.sum(p, axis=0, keepdims=True) * (1.0 / S)
    aux_ref[...] = jnp.sum(counts * pmean, axis=1, keepdims=True) * (E / S)


def _router_call(h1, Wg, g, b):
    return pl.pallas_call(
        _router_body,
        grid=(1,),
        in_specs=[
            pl.BlockSpec((S, H), lambda i: (0, 0)),
            pl.BlockSpec((H, E), lambda i: (0, 0)),
            pl.BlockSpec((1, H), lambda i: (0, 0)),
            pl.BlockSpec((1, H), lambda i: (0, 0)),
        ],
        out_specs=[
            pl.BlockSpec((S, H), lambda i: (0, 0)),
            pl.BlockSpec((S, 2), lambda i: (0, 0)),
            pl.BlockSpec((S, 2), lambda i: (0, 0)),
            pl.BlockSpec((8, 64), lambda i: (0, 0)),
            pl.BlockSpec((1, 1), lambda i: (0, 0)),
        ],
        out_shape=[
            jax.ShapeDtypeStruct((S, H), _f32),
            jax.ShapeDtypeStruct((S, 2), _f32),
            jax.ShapeDtypeStruct((S, 2), jnp.int32),
            jax.ShapeDtypeStruct((8, 64), jnp.int32),
            jax.ShapeDtypeStruct((1, 1), _f32),
        ],
        scratch_shapes=[pltpu.VMEM((2 * S, 1), _f32),
                        pltpu.VMEM((2 * S, 1), jnp.int32)],
    )(h1, Wg, g, b)


def _sc_mesh():
    return plsc.VectorSubcoreMesh(core_axis_name="c", subcore_axis_name="s")


NW = 32
TPW = S // NW


def _dispatch_body(x_hbm, d0_hbm, d1_hbm, xs_hbm, idx_v, rows_v, sem):
    wid = lax.axis_index("s") * 2 + lax.axis_index("c")
    base = wid * TPW
    pltpu.sync_copy(x_hbm.at[pl.ds(base, TPW)], rows_v)
    pltpu.sync_copy(d0_hbm.at[pl.ds(base, TPW)], idx_v)
    pltpu.async_copy(rows_v, xs_hbm.at[idx_v], sem).wait()
    pltpu.sync_copy(d1_hbm.at[pl.ds(base, TPW)], idx_v)
    pltpu.async_copy(rows_v, xs_hbm.at[idx_v], sem).wait()


def _dispatch(xn2, d0, d1):
    k = pl.kernel(
        _dispatch_body,
        out_type=jax.ShapeDtypeStruct((LPAD, H), _f32),
        mesh=_sc_mesh(),
        scratch_types=[
            pltpu.VMEM((TPW,), jnp.int32),
            pltpu.VMEM((TPW, H), _f32),
            pltpu.SemaphoreType.DMA,
        ],
    )
    return k(xn2, d0, d1)


def _combine_body(ye_hbm, d0_hbm, d1_hbm, y0_hbm, y1_hbm, idx_v, rows_v, sem):
    wid = lax.axis_index("s") * 2 + lax.axis_index("c")
    base = wid * TPW
    pltpu.sync_copy(d0_hbm.at[pl.ds(base, TPW)], idx_v)
    pltpu.async_copy(ye_hbm.at[idx_v], rows_v, sem).wait()
    pltpu.sync_copy(rows_v, y0_hbm.at[pl.ds(base, TPW)])
    pltpu.sync_copy(d1_hbm.at[pl.ds(base, TPW)], idx_v)
    pltpu.async_copy(ye_hbm.at[idx_v], rows_v, sem).wait()
    pltpu.sync_copy(rows_v, y1_hbm.at[pl.ds(base, TPW)])


def _combine(ye, d0, d1):
    sd = jax.ShapeDtypeStruct((S, H), _f32)
    k = pl.kernel(
        _combine_body,
        out_type=[sd, sd],
        mesh=_sc_mesh(),
        scratch_types=[
            pltpu.VMEM((TPW,), jnp.int32),
            pltpu.VMEM((TPW, H), _f32),
            pltpu.SemaphoreType.DMA,
        ],
    )
    return k(ye, d0, d1)


def _gateup_body(be_ref, xs_ref, wg_ref, wu_ref, z_ref):
    x = xs_ref[...].astype(_bf16)
    g = jnp.dot(x, wg_ref[0].astype(_bf16), preferred_element_type=_f32)
    u = jnp.dot(x, wu_ref[0].astype(_bf16), preferred_element_type=_f32)
    z_ref[...] = (jax.nn.silu(g) * u).astype(_bf16)


def _gateup_call(be, xs, W_gate, W_up):
    gs = pltpu.PrefetchScalarGridSpec(
        num_scalar_prefetch=1,
        grid=(NBLK,),
        in_specs=[
            pl.BlockSpec((BM, H), lambda i, be: (i, 0)),
            pl.BlockSpec((1, H, FF), lambda i, be: (be[i], 0, 0)),
            pl.BlockSpec((1, H, FF), lambda i, be: (be[i], 0, 0)),
        ],
        out_specs=pl.BlockSpec((BM, FF), lambda i, be: (i, 0)),
    )
    return pl.pallas_call(
        _gateup_body,
        grid_spec=gs,
        out_shape=jax.ShapeDtypeStruct((LPAD, FF), _bf16),
    )(be, xs, W_gate, W_up)


def _down_body(be_ref, z_ref, wd_ref, y_ref):
    y_ref[...] = jnp.dot(z_ref[...], wd_ref[0].astype(_bf16),
                         preferred_element_type=_f32)


def _down_call(be, z, W_down):
    gs = pltpu.PrefetchScalarGridSpec(
        num_scalar_prefetch=1,
        grid=(NBLK,),
        in_specs=[
            pl.BlockSpec((BM, FF), lambda i, be: (i, 0)),
            pl.BlockSpec((1, FF, H), lambda i, be: (be[i], 0, 0)),
        ],
        out_specs=pl.BlockSpec((BM, H), lambda i, be: (i, 0)),
    )
    return pl.pallas_call(
        _down_body,
        grid_spec=gs,
        out_shape=jax.ShapeDtypeStruct((LPAD, H), _f32),
    )(be, z, W_down)


def _final_body(h_ref, y0_ref, y1_ref, w_ref, o_ref):
    o_ref[...] = (h_ref[...]
                  + w_ref[:, 0:1] * y0_ref[...]
                  + w_ref[:, 1:2] * y1_ref[...])


def _final_call(h1, y0, y1, wts):
    return pl.pallas_call(
        _final_body,
        grid=(S // BS,),
        in_specs=[
            pl.BlockSpec((BS, H), lambda i: (i, 0)),
            pl.BlockSpec((BS, H), lambda i: (i, 0)),
            pl.BlockSpec((BS, H), lambda i: (i, 0)),
            pl.BlockSpec((BS, 2), lambda i: (i, 0)),
        ],
        out_specs=pl.BlockSpec((BS, H), lambda i: (i, 0)),
        out_shape=jax.ShapeDtypeStruct((S, H), _f32),
    )(h1, y0, y1, wts)


def kernel(hidden_states, Wq, Wk, Wv, Wo, Wg, W_gate, W_up, W_down,
           ln1_w, ln1_b, ln2_w, ln2_b):
    x0 = hidden_states.reshape(S, H)
    g1 = ln1_w.reshape(1, H)
    b1 = ln1_b.reshape(1, H)
    g2 = ln2_w.reshape(1, H)
    b2 = ln2_b.reshape(1, H)

    pos = jnp.arange(S, dtype=_f32)
    inv = 1.0 / (10000.0 ** (jnp.arange(0, HD, 2, dtype=_f32) / HD))
    fr = jnp.outer(pos, inv)
    cos = jnp.concatenate([jnp.cos(fr), jnp.cos(fr)], axis=-1)
    sin = jnp.concatenate([jnp.sin(fr), jnp.sin(fr)], axis=-1)

    q, k, v = _qkv_call(x0, Wq, Wk, Wv, g1, b1)
    qh = q.reshape(S, NH, HD).transpose(1, 0, 2)
    kh = k.reshape(S, NH, HD).transpose(1, 0, 2)
    vh = v.reshape(S, NH, HD).transpose(1, 0, 2)
    attn = _attn_call(qh, kh, vh, cos, sin)
    attn2 = attn.transpose(1, 0, 2).reshape(S, H)
    h1 = _oproj_call(attn2, Wo, x0)

    xn2, wts, dest, blk, aux = _router_call(h1, Wg, g2, b2)
    d0 = dest[:, 0]
    d1 = dest[:, 1]
    be = blk[0, :NBLK]

    xs = _dispatch(xn2, d0, d1)
    z = _gateup_call(be, xs, W_gate, W_up)
    ye = _down_call(be, z, W_down)
    y0, y1 = _combine(ye, d0, d1)
    out = _final_call(h1, y0, y1, wts)
    return out.reshape(B, S, H), aux[0, 0]
